# Initial kernel scaffold; baseline (speedup 1.0000x reference)
import jax
import jax.numpy as jnp
from jax import lax
from jax.experimental import pallas as pl
from jax.experimental.pallas import tpu as pltpu

NUM_EMB = 1024
EMB_DIM = 64
GROUPS = 16
TOTAL_DIM = GROUPS * EMB_DIM
BN = 256


def _vq_kernel(in_ref, w_ref, out_ref, idx_ref, loss_ref, wsq_ref):
    pid = pl.program_id(0)
    nblocks = pl.num_programs(0)

    @pl.when(pid == 0)
    def _():
        for g in range(GROUPS):
            wg = w_ref[g]
            wsq_ref[g, :] = jnp.sum(wg * wg, axis=1)

    loss_part = jnp.float32(0.0)
    for g in range(GROUPS):
        v = in_ref[:, g * EMB_DIM:(g + 1) * EMB_DIM]
        norm = jnp.sqrt(jnp.sum(v * v, axis=1, keepdims=True))
        xg = v / (norm + 1e-6)
        x_sq = jnp.sum(xg * xg, axis=1, keepdims=True)
        wg = w_ref[g]
        scores = lax.dot_general(
            xg, wg, (((1,), (1,)), ((), ())),
            preferred_element_type=jnp.float32)
        dist = (x_sq + wsq_ref[g:g + 1, :]) - 2.0 * scores
        minv = jnp.min(dist, axis=1, keepdims=True)
        iota = lax.broadcasted_iota(jnp.int32, (BN, NUM_EMB), 1)
        idx_col = jnp.min(jnp.where(dist == minv, iota, NUM_EMB),
                          axis=1, keepdims=True)
        idx_ref[:, g:g + 1] = idx_col
        onehot = (iota == idx_col).astype(jnp.float32)
        q = lax.dot_general(
            onehot, wg, (((1,), (0,)), ((), ())),
            preferred_element_type=jnp.float32)
        d_qx = q - xg
        out_ref[:, g * EMB_DIM:(g + 1) * EMB_DIM] = xg + d_qx
        loss_part = loss_part + jnp.sum(d_qx * d_qx)

    prev = jnp.where(pid == 0, 0.0, loss_ref[0, 0])
    acc = prev + loss_part
    scale = 1.25 / (GROUPS * 9216 * EMB_DIM)
    loss_ref[0, 0] = jnp.where(pid == nblocks - 1, acc * scale, acc)


def kernel(inputs, embed_weights):
    input_shape = inputs.shape
    flat = inputs.reshape(-1, TOTAL_DIM)
    n = flat.shape[0]
    nblocks = n // BN

    out, idx, loss = pl.pallas_call(
        _vq_kernel,
        grid=(nblocks,),
        in_specs=[
            pl.BlockSpec((BN, TOTAL_DIM), lambda i: (i, 0)),
            pl.BlockSpec((GROUPS, NUM_EMB, EMB_DIM), lambda i: (0, 0, 0)),
        ],
        out_specs=[
            pl.BlockSpec((BN, TOTAL_DIM), lambda i: (i, 0)),
            pl.BlockSpec((BN, GROUPS), lambda i: (i, 0)),
            pl.BlockSpec((1, 1), lambda i: (0, 0)),
        ],
        out_shape=[
            jax.ShapeDtypeStruct((n, TOTAL_DIM), jnp.float32),
            jax.ShapeDtypeStruct((n, GROUPS), jnp.int32),
            jax.ShapeDtypeStruct((1, 1), jnp.float32),
        ],
        scratch_shapes=[pltpu.VMEM((GROUPS, NUM_EMB), jnp.float32)],
    )(flat, embed_weights)

    quantized_out = out.reshape(input_shape)
    indices_out = idx.reshape(*input_shape[:-1], GROUPS)
    total_loss = loss[0, 0]
    return (quantized_out, total_loss, indices_out)


# speedup vs baseline: 5.8630x; 5.8630x over previous
import jax
import jax.numpy as jnp
from jax import lax
from jax.experimental import pallas as pl

NUM_EMB = 1024
EMB_DIM = 64
GROUPS = 16
TOTAL_DIM = GROUPS * EMB_DIM
BN = 256


def _vq_kernel(x_ref, xsq_ref, w_ref, wsq_ref, out_ref, idx_ref, loss_ref):
    pid = pl.program_id(0)
    nblocks = pl.num_programs(0)

    loss_part = jnp.float32(0.0)
    for g in range(GROUPS):
        xg = x_ref[:, g * EMB_DIM:(g + 1) * EMB_DIM]
        wg = w_ref[g]
        scores = lax.dot_general(
            xg, wg, (((1,), (1,)), ((), ())),
            preferred_element_type=jnp.float32)
        dist = (xsq_ref[:, g:g + 1] + wsq_ref[g:g + 1, :]) - 2.0 * scores
        minv = jnp.min(dist, axis=1, keepdims=True)
        iota = lax.broadcasted_iota(jnp.int32, (BN, NUM_EMB), 1)
        idx_col = jnp.min(jnp.where(dist == minv, iota, NUM_EMB),
                          axis=1, keepdims=True)
        idx_ref[:, g:g + 1] = idx_col
        onehot = (iota == idx_col).astype(jnp.float32)
        q = lax.dot_general(
            onehot, wg, (((1,), (0,)), ((), ())),
            preferred_element_type=jnp.float32)
        d_qx = q - xg
        out_ref[:, g * EMB_DIM:(g + 1) * EMB_DIM] = xg + d_qx
        loss_part = loss_part + jnp.sum(d_qx * d_qx)

    prev = jnp.where(pid == 0, jnp.zeros((1, 1), jnp.float32), loss_ref[:, :])
    acc = prev + loss_part
    scale = 1.25 / (GROUPS * 9216 * EMB_DIM)
    loss_ref[:, :] = jnp.where(pid == nblocks - 1, acc * scale, acc)


def kernel(inputs, embed_weights):
    input_shape = inputs.shape
    flat = inputs.reshape(-1, TOTAL_DIM)
    n = flat.shape[0]
    nblocks = n // BN

    grouped = flat.reshape(-1, GROUPS, EMB_DIM)
    norms = jnp.linalg.norm(grouped, axis=2, keepdims=True)
    x = grouped / (norms + 1e-6)
    x_sq = jnp.sum(x**2, axis=2)
    w_sq = jnp.sum(embed_weights**2, axis=2)
    x2d = x.reshape(n, TOTAL_DIM)

    out, idx, loss = pl.pallas_call(
        _vq_kernel,
        grid=(nblocks,),
        in_specs=[
            pl.BlockSpec((BN, TOTAL_DIM), lambda i: (i, 0)),
            pl.BlockSpec((BN, GROUPS), lambda i: (i, 0)),
            pl.BlockSpec((GROUPS, NUM_EMB, EMB_DIM), lambda i: (0, 0, 0)),
            pl.BlockSpec((GROUPS, NUM_EMB), lambda i: (0, 0)),
        ],
        out_specs=[
            pl.BlockSpec((BN, TOTAL_DIM), lambda i: (i, 0)),
            pl.BlockSpec((BN, GROUPS), lambda i: (i, 0)),
            pl.BlockSpec((1, 1), lambda i: (0, 0)),
        ],
        out_shape=[
            jax.ShapeDtypeStruct((n, TOTAL_DIM), jnp.float32),
            jax.ShapeDtypeStruct((n, GROUPS), jnp.int32),
            jax.ShapeDtypeStruct((1, 1), jnp.float32),
        ],
    )(x2d, x_sq, embed_weights, w_sq)

    quantized_out = out.reshape(input_shape)
    indices_out = idx.reshape(*input_shape[:-1], GROUPS)
    total_loss = loss[0, 0]
    return (quantized_out, total_loss, indices_out)


# speedup vs baseline: 5.8646x; 1.0003x over previous
import jax
import jax.numpy as jnp
from jax import lax
from jax.experimental import pallas as pl

NUM_EMB = 1024
EMB_DIM = 64
GROUPS = 16
TOTAL_DIM = GROUPS * EMB_DIM
BN = 256


def _vq_kernel(x_ref, xsq_ref, w_ref, wsq_ref, out_ref, idx_ref, loss_ref):
    pid = pl.program_id(0)
    nblocks = pl.num_programs(0)

    loss_part = jnp.float32(0.0)
    for g in range(GROUPS):
        xg = x_ref[:, g * EMB_DIM:(g + 1) * EMB_DIM]
        wg = w_ref[g]
        scores = lax.dot_general(
            xg, wg, (((1,), (1,)), ((), ())),
            preferred_element_type=jnp.float32)
        dist = (xsq_ref[:, g:g + 1] + wsq_ref[g:g + 1, :]) - 2.0 * scores
        minv = jnp.min(dist, axis=1, keepdims=True)
        iota = lax.broadcasted_iota(jnp.int32, (BN, NUM_EMB), 1)
        idx_col = jnp.min(jnp.where(dist == minv, iota, NUM_EMB),
                          axis=1, keepdims=True)
        idx_ref[:, g:g + 1] = idx_col
        onehot = (iota == idx_col).astype(jnp.float32)
        q = lax.dot_general(
            onehot, wg, (((1,), (0,)), ((), ())),
            preferred_element_type=jnp.float32)
        d_qx = q - xg
        out_ref[:, g * EMB_DIM:(g + 1) * EMB_DIM] = xg + d_qx
        loss_part = loss_part + jnp.sum(d_qx * d_qx)

    prev = jnp.where(pid == 0, jnp.zeros((1, 1), jnp.float32), loss_ref[:, :])
    acc = prev + loss_part
    scale = 1.25 / (GROUPS * 9216 * EMB_DIM)
    loss_ref[:, :] = jnp.where(pid == nblocks - 1, acc * scale, acc)


def kernel(inputs, embed_weights):
    input_shape = inputs.shape
    flat = inputs.reshape(-1, TOTAL_DIM)
    n = flat.shape[0]
    nblocks = n // BN

    grouped = flat.reshape(-1, GROUPS, EMB_DIM)
    norms = jnp.linalg.norm(grouped, axis=2, keepdims=True)
    den = jnp.repeat((norms + 1e-6).reshape(n, GROUPS), EMB_DIM, axis=1)
    x2d = flat / den
    x_sq = jnp.sum(x2d.reshape(-1, GROUPS, EMB_DIM) ** 2, axis=2)
    w_sq = jnp.sum(embed_weights**2, axis=2)

    out, idx, loss = pl.pallas_call(
        _vq_kernel,
        grid=(nblocks,),
        in_specs=[
            pl.BlockSpec((BN, TOTAL_DIM), lambda i: (i, 0)),
            pl.BlockSpec((BN, GROUPS), lambda i: (i, 0)),
            pl.BlockSpec((GROUPS, NUM_EMB, EMB_DIM), lambda i: (0, 0, 0)),
            pl.BlockSpec((GROUPS, NUM_EMB), lambda i: (0, 0)),
        ],
        out_specs=[
            pl.BlockSpec((BN, TOTAL_DIM), lambda i: (i, 0)),
            pl.BlockSpec((BN, GROUPS), lambda i: (i, 0)),
            pl.BlockSpec((1, 1), lambda i: (0, 0)),
        ],
        out_shape=[
            jax.ShapeDtypeStruct((n, TOTAL_DIM), jnp.float32),
            jax.ShapeDtypeStruct((n, GROUPS), jnp.int32),
            jax.ShapeDtypeStruct((1, 1), jnp.float32),
        ],
    )(x2d, x_sq, embed_weights, w_sq)

    quantized_out = out.reshape(input_shape)
    indices_out = idx.reshape(*input_shape[:-1], GROUPS)
    total_loss = loss[0, 0]
    return (quantized_out, total_loss, indices_out)
